# baseline (speedup 1.0000x reference)
import functools
import math

import jax
import jax.numpy as jnp
from jax import lax
from jax.experimental import pallas as pl
from jax.experimental.pallas import tpu as pltpu
from jax.experimental.pallas import tpu_sc as plsc

TOP_K = 10
SID_PREFIX = 4
L = 16

_GAINS = [1.0 / math.log2(r + 2) for r in range(TOP_K)]
_GAINV = _GAINS + [0.0] * (L - TOP_K)
_IDCGV = [sum(_GAINS[:m]) for m in range(TOP_K + 1)] + [0.0] * (L - TOP_K - 1)


def _splat(x, dtype=jnp.int32):
    return jnp.full((L,), x, dtype)


def _make_sc_call(B, C):
    NC, NS = 2, 16
    NW = NC * NS
    ROWS_W = B // NW
    RB = 8
    NBLK = ROWS_W // RB
    NCH = (C + L - 1) // L
    CP = NCH * L
    WPC = 2 * SID_PREFIX
    IDS_W = C * WPC
    IDS_WP = CP * WPC
    LABW = 2 * SID_PREFIX

    mesh = plsc.VectorSubcoreMesh(
        core_axis_name="c", subcore_axis_name="s",
        num_cores=NC, num_subcores=NS)

    @functools.partial(
        pl.kernel,
        mesh=mesh,
        compiler_params=pltpu.CompilerParams(
            use_tc_tiling_on_sc=False, needs_layout_passes=False),
        out_type=jax.ShapeDtypeStruct((NC, L), jnp.float32),
        scratch_types=[
            pltpu.VMEM((RB, CP), jnp.float32),
            pltpu.VMEM((RB, IDS_WP), jnp.int32),
            pltpu.VMEM((RB * LABW + L,), jnp.int32),
            pltpu.VMEM((3 * L,), jnp.float32),
            pltpu.VMEM_SHARED((NS, 3 * L), jnp.float32),
            pltpu.VMEM((NS, 3 * L), jnp.float32),
            pltpu.VMEM((L,), jnp.float32),
            pltpu.VMEM((L,), jnp.float32),
            pltpu.VMEM((L,), jnp.float32),
        ],
    )
    def sc_call(lp_hbm, ids_hbm, lab_hbm, gain_hbm, idcg_hbm, out_hbm,
                lp_v, ids_v, lab_v, part_v, shared_v, red_v, res_v,
                gain_v, idcg_v):
        cid = lax.axis_index("c")
        sid = lax.axis_index("s")
        wid = sid * NC + cid
        base0 = wid * ROWS_W

        iota = lax.iota(jnp.int32, L)
        iota8 = iota * WPC
        pltpu.sync_copy(gain_hbm, gain_v)
        pltpu.sync_copy(idcg_hbm, idcg_v)

        neg_inf = _splat(-jnp.inf, jnp.float32)
        neg_one = _splat(-1, jnp.int32)
        for r in range(RB):
            lp_v[r, pl.ds(CP - L, L)] = neg_inf
            for q in range(IDS_W // L, IDS_WP // L):
                ids_v[r, pl.ds(q * L, L)] = neg_one
        lab_v[pl.ds(RB * LABW, L)] = _splat(0, jnp.int32)

        def block_body(g, accs):
            base = base0 + g * RB
            pltpu.sync_copy(lp_hbm.at[pl.ds(base, RB)], lp_v.at[:, pl.ds(0, C)])
            pltpu.sync_copy(ids_hbm.at[pl.ds(base, RB)],
                            ids_v.at[:, pl.ds(0, IDS_W)])
            pltpu.sync_copy(lab_hbm.at[pl.ds(base * LABW, RB * LABW)],
                            lab_v.at[pl.ds(0, RB * LABW)])

            def row_body(r, accs2):
                acc_nd, acc_rc, acc_ht = accs2
                lw = [plsc.load_gather(lab_v, [_splat(r * LABW + 2 * h)])
                      for h in range(SID_PREFIX)]
                rsplat = _splat(r)

                def chunk_mask(j):
                    m = None
                    for h in range(SID_PREFIX):
                        widx = _splat(j * L * WPC + 2 * h) + iota8
                        g_h = plsc.load_gather(ids_v, [rsplat, widx])
                        d = g_h ^ lw[h]
                        m = d if m is None else (m | d)
                    return m == 0

                trel = _splat(0, jnp.int32)
                for j in range(NCH):
                    trel = trel + plsc.all_reduce_population_count(chunk_mask(j))

                def rare(_):
                    dcg = _splat(0.0, jnp.float32)
                    nh = _splat(0, jnp.int32)

                    def wcond(carry):
                        mm, _, _ = carry
                        return jnp.max(mm.astype(jnp.int32)) > 0

                    for j in range(NCH):
                        def wbody(carry, j=j):
                            mm, dcg_, nh_ = carry
                            ffs = plsc.all_reduce_ffs(mm)
                            bp = plsc.load_gather(
                                lp_v, [rsplat, _splat(j * L) + ffs])
                            bi = _splat(j * L) + ffs
                            cnt = _splat(0, jnp.int32)
                            for k in range(NCH):
                                pk = lp_v[r, pl.ds(k * L, L)]
                                ik = _splat(k * L) + iota
                                better = (pk > bp) | ((pk == bp) & (ik < bi))
                                cnt = cnt + plsc.all_reduce_population_count(
                                    better)
                            gain = plsc.load_gather(
                                gain_v, [jnp.minimum(cnt, TOP_K)])
                            dcg_ = dcg_ + gain
                            nh_ = nh_ + jnp.where(
                                cnt < TOP_K, jnp.int32(1), jnp.int32(0))
                            mm = mm & (iota != ffs)
                            return mm, dcg_, nh_

                        _, dcg, nh = lax.while_loop(
                            wcond, wbody, (chunk_mask(j), dcg, nh))

                    rm = jnp.minimum(trel, TOP_K)
                    idcg = plsc.load_gather(idcg_v, [rm])
                    nd = dcg / idcg
                    rc = nh.astype(jnp.float32) / rm.astype(jnp.float32)
                    ht = jnp.where(nh > 0, jnp.float32(1.0), jnp.float32(0.0))
                    return nd, rc, ht

                def none(_):
                    z = _splat(0.0, jnp.float32)
                    return z, z, z

                has_rel = jnp.max(trel) > 0
                nd, rc, ht = lax.cond(has_rel, rare, none, None)
                return acc_nd + nd, acc_rc + rc, acc_ht + ht

            return lax.fori_loop(jnp.int32(0), jnp.int32(RB), row_body, accs)

        zf = _splat(0.0, jnp.float32)
        acc_nd, acc_rc, acc_ht = lax.fori_loop(
            jnp.int32(0), jnp.int32(NBLK), block_body, (zf, zf, zf))

        part_v[pl.ds(0, L)] = acc_nd
        part_v[pl.ds(L, L)] = acc_rc
        part_v[pl.ds(2 * L, L)] = acc_ht
        pltpu.sync_copy(part_v, shared_v.at[sid])
        plsc.subcore_barrier()

        @pl.when(sid == 0)
        def _():
            pltpu.sync_copy(shared_v, red_v)
            snd = zf
            src = zf
            sht = zf
            for i in range(NS):
                snd = snd + red_v[i, pl.ds(0, L)]
                src = src + red_v[i, pl.ds(L, L)]
                sht = sht + red_v[i, pl.ds(2 * L, L)]
            inv_b = jnp.float32(1.0 / B)
            res = (jnp.where(iota == 0, snd, 0.0)
                   + jnp.where(iota == 1, src, 0.0)
                   + jnp.where(iota == 2, sht, 0.0)) * inv_b
            res_v[pl.ds(0, L)] = res
            pltpu.sync_copy(res_v, out_hbm.at[cid])

    return sc_call


def kernel(log_probs, generated_ids, labels):
    B, C, H = generated_ids.shape
    gi = lax.bitcast_convert_type(
        generated_ids[:, :, :SID_PREFIX], jnp.int32).reshape(B, C * 2 * SID_PREFIX)
    lab = lax.bitcast_convert_type(
        labels[:, :SID_PREFIX], jnp.int32).reshape(B * 2 * SID_PREFIX)
    lp = log_probs.astype(jnp.float32)
    gain_tab = jnp.asarray(_GAINV, jnp.float32)
    idcg_tab = jnp.asarray(_IDCGV, jnp.float32)
    out = _make_sc_call(B, C)(lp, gi, lab, gain_tab, idcg_tab)
    s = out[0] + out[1]
    return (s[0], s[1], s[2])


# speedup vs baseline: 2.2941x; 2.2941x over previous
import functools
import math

import jax
import jax.numpy as jnp
from jax import lax
from jax.experimental import pallas as pl
from jax.experimental.pallas import tpu as pltpu
from jax.experimental.pallas import tpu_sc as plsc

TOP_K = 10
SID_PREFIX = 4
L = 16

_GAINS = [1.0 / math.log2(r + 2) for r in range(TOP_K)]
_GAINV = _GAINS + [0.0] * (L - TOP_K)
_IDCGV = [sum(_GAINS[:m]) for m in range(TOP_K + 1)] + [0.0] * (L - TOP_K - 1)


def _splat(x, dtype=jnp.int32):
    return jnp.full((L,), x, dtype)


def _make_sc_call(B, C):
    NC, NS = 2, 16
    NW = NC * NS
    ROWS_W = B // NW
    RB = 8
    NBLK = ROWS_W // RB
    NCH = (C + L - 1) // L
    CP = NCH * L
    WPC = SID_PREFIX
    IDS_W = C * WPC
    IDS_WP = CP * WPC
    LABW = SID_PREFIX

    mesh = plsc.VectorSubcoreMesh(
        core_axis_name="c", subcore_axis_name="s",
        num_cores=NC, num_subcores=NS)

    @functools.partial(
        pl.kernel,
        mesh=mesh,
        compiler_params=pltpu.CompilerParams(
            use_tc_tiling_on_sc=False, needs_layout_passes=False),
        out_type=jax.ShapeDtypeStruct((NC, L), jnp.float32),
        scratch_types=[
            pltpu.VMEM((RB, CP), jnp.float32),
            pltpu.VMEM((RB, CP), jnp.float32),
            pltpu.VMEM((RB, IDS_WP), jnp.int32),
            pltpu.VMEM((RB, IDS_WP), jnp.int32),
            pltpu.VMEM((ROWS_W * LABW + L,), jnp.int32),
            pltpu.VMEM((3 * L,), jnp.float32),
            pltpu.VMEM_SHARED((NS, 3 * L), jnp.float32),
            pltpu.VMEM((NS, 3 * L), jnp.float32),
            pltpu.VMEM((L,), jnp.float32),
            pltpu.VMEM((L,), jnp.float32),
            pltpu.VMEM((L,), jnp.float32),
            pltpu.SemaphoreType.DMA,
            pltpu.SemaphoreType.DMA,
        ],
    )
    def sc_call(lp_hbm, ids_hbm, lab_hbm, gain_hbm, idcg_hbm, out_hbm,
                lp_v0, lp_v1, ids_v0, ids_v1, lab_v, part_v, shared_v,
                red_v, res_v, gain_v, idcg_v, sem0, sem1):
        cid = lax.axis_index("c")
        sid = lax.axis_index("s")
        wid = sid * NC + cid
        base0 = wid * ROWS_W

        iota = lax.iota(jnp.int32, L)
        iota8 = iota * WPC
        pltpu.sync_copy(gain_hbm, gain_v)
        pltpu.sync_copy(idcg_hbm, idcg_v)
        pltpu.sync_copy(lab_hbm.at[pl.ds(base0 * LABW, ROWS_W * LABW)],
                        lab_v.at[pl.ds(0, ROWS_W * LABW)])
        lab_v[pl.ds(ROWS_W * LABW, L)] = _splat(0, jnp.int32)

        neg_inf = _splat(-jnp.inf, jnp.float32)
        neg_one = _splat(-1, jnp.int32)
        for lp_b, ids_b in ((lp_v0, ids_v0), (lp_v1, ids_v1)):
            for r in range(RB):
                lp_b[r, pl.ds(CP - L, L)] = neg_inf
                for q in range(IDS_W // L, IDS_WP // L):
                    ids_b[r, pl.ds(q * L, L)] = neg_one

        def blk_copies(g, lp_b, ids_b, sem):
            base = base0 + g * RB
            return (
                pltpu.make_async_copy(
                    lp_hbm.at[pl.ds(base, RB)], lp_b.at[:, pl.ds(0, C)], sem),
                pltpu.make_async_copy(
                    ids_hbm.at[pl.ds(base, RB)],
                    ids_b.at[:, pl.ds(0, IDS_W)], sem),
            )

        def start_blk(g, lp_b, ids_b, sem):
            for c in blk_copies(g, lp_b, ids_b, sem):
                c.start()

        def wait_blk(g, lp_b, ids_b, sem):
            for c in blk_copies(g, lp_b, ids_b, sem):
                c.wait()

        def process_block(g, lp_v, ids_v, accs):
            def row_body(r, accs2):
                acc_nd, acc_rc, acc_ht = accs2
                lwoff = (g * RB + r) * LABW
                lw = [plsc.load_gather(lab_v, [_splat(lwoff + h)])
                      for h in range(SID_PREFIX)]
                rsplat = _splat(r)

                def chunk_mask(j):
                    m = None
                    for h in range(SID_PREFIX):
                        widx = _splat(j * L * WPC + h) + iota8
                        g_h = plsc.load_gather(ids_v, [rsplat, widx])
                        d = g_h ^ lw[h]
                        m = d if m is None else (m | d)
                    return m == 0

                trel = _splat(0, jnp.int32)
                for j in range(NCH):
                    trel = trel + plsc.all_reduce_population_count(chunk_mask(j))

                def rare(_):
                    dcg = _splat(0.0, jnp.float32)
                    nh = _splat(0, jnp.int32)

                    def wcond(carry):
                        mm, _, _ = carry
                        return jnp.max(mm.astype(jnp.int32)) > 0

                    for j in range(NCH):
                        def wbody(carry, j=j):
                            mm, dcg_, nh_ = carry
                            ffs = plsc.all_reduce_ffs(mm)
                            bp = plsc.load_gather(
                                lp_v, [rsplat, _splat(j * L) + ffs])
                            bi = _splat(j * L) + ffs
                            cnt = _splat(0, jnp.int32)
                            for k in range(NCH):
                                pk = lp_v[r, pl.ds(k * L, L)]
                                ik = _splat(k * L) + iota
                                better = (pk > bp) | ((pk == bp) & (ik < bi))
                                cnt = cnt + plsc.all_reduce_population_count(
                                    better)
                            gain = plsc.load_gather(
                                gain_v, [jnp.minimum(cnt, TOP_K)])
                            dcg_ = dcg_ + gain
                            nh_ = nh_ + jnp.where(
                                cnt < TOP_K, jnp.int32(1), jnp.int32(0))
                            mm = mm & (iota != ffs)
                            return mm, dcg_, nh_

                        _, dcg, nh = lax.while_loop(
                            wcond, wbody, (chunk_mask(j), dcg, nh))

                    rm = jnp.minimum(trel, TOP_K)
                    idcg = plsc.load_gather(idcg_v, [rm])
                    nd = dcg / idcg
                    rc = nh.astype(jnp.float32) / rm.astype(jnp.float32)
                    ht = jnp.where(nh > 0, jnp.float32(1.0), jnp.float32(0.0))
                    return nd, rc, ht

                def none(_):
                    z = _splat(0.0, jnp.float32)
                    return z, z, z

                has_rel = jnp.max(trel) > 0
                nd, rc, ht = lax.cond(has_rel, rare, none, None)
                return acc_nd + nd, acc_rc + rc, acc_ht + ht

            return lax.fori_loop(jnp.int32(0), jnp.int32(RB), row_body, accs)

        start_blk(jnp.int32(0), lp_v0, ids_v0, sem0)

        def big_body(gg, accs):
            g0 = gg * 2
            start_blk(g0 + 1, lp_v1, ids_v1, sem1)
            wait_blk(g0, lp_v0, ids_v0, sem0)
            accs = process_block(g0, lp_v0, ids_v0, accs)

            @pl.when(g0 + 2 < NBLK)
            def _():
                start_blk(g0 + 2, lp_v0, ids_v0, sem0)

            wait_blk(g0 + 1, lp_v1, ids_v1, sem1)
            return process_block(g0 + 1, lp_v1, ids_v1, accs)

        zf = _splat(0.0, jnp.float32)
        acc_nd, acc_rc, acc_ht = lax.fori_loop(
            jnp.int32(0), jnp.int32(NBLK // 2), big_body, (zf, zf, zf))

        part_v[pl.ds(0, L)] = acc_nd
        part_v[pl.ds(L, L)] = acc_rc
        part_v[pl.ds(2 * L, L)] = acc_ht
        pltpu.sync_copy(part_v, shared_v.at[sid])
        plsc.subcore_barrier()

        @pl.when(sid == 0)
        def _():
            pltpu.sync_copy(shared_v, red_v)
            snd = zf
            src = zf
            sht = zf
            for i in range(NS):
                snd = snd + red_v[i, pl.ds(0, L)]
                src = src + red_v[i, pl.ds(L, L)]
                sht = sht + red_v[i, pl.ds(2 * L, L)]
            inv_b = jnp.float32(1.0 / B)
            res = (jnp.where(iota == 0, snd, 0.0)
                   + jnp.where(iota == 1, src, 0.0)
                   + jnp.where(iota == 2, sht, 0.0)) * inv_b
            res_v[pl.ds(0, L)] = res
            pltpu.sync_copy(res_v, out_hbm.at[cid])

    return sc_call


def kernel(log_probs, generated_ids, labels):
    B, C, H = generated_ids.shape
    gi = generated_ids[:, :, :SID_PREFIX].astype(jnp.int32).reshape(
        B, C * SID_PREFIX)
    lab = labels[:, :SID_PREFIX].astype(jnp.int32).reshape(B * SID_PREFIX)
    lp = log_probs.astype(jnp.float32)
    gain_tab = jnp.asarray(_GAINV, jnp.float32)
    idcg_tab = jnp.asarray(_IDCGV, jnp.float32)
    out = _make_sc_call(B, C)(lp, gi, lab, gain_tab, idcg_tab)
    s = out[0] + out[1]
    return (s[0], s[1], s[2])
